# Initial kernel scaffold; baseline (speedup 1.0000x reference)
import jax
import jax.numpy as jnp
from jax.experimental import pallas as pl


def kernel(x, edge_index, batch, emb, W1, b1, W2, b2, Wlin, blin):
    raise NotImplementedError("write your pallas kernel here")


# speedup vs baseline: 6.9441x; 6.9441x over previous
import functools

import jax
import jax.numpy as jnp
from jax import lax
from jax.experimental import pallas as pl
from jax.experimental.pallas import tpu as pltpu
from jax.experimental.pallas import tpu_sc as plsc

N = 50000
NPAD = 50176
NHALF = NPAD // 2
RPT = NPAD // 32
ZROWS = NPAD - N
E = 800000
ETOT = E + N
EPAD = 851968
EPT = EPAD // 16
W = 128
D_EMB = 64
D_H = 128
VOCAB = 1024
NG = 256
BM = 512
NBLK = NPAD // BM

_mesh = plsc.VectorSubcoreMesh(core_axis_name="c", subcore_axis_name="s")
_sc_params = pltpu.CompilerParams(needs_layout_passes=False)
_sc_params_untiled = pltpu.CompilerParams(needs_layout_passes=False,
                                          use_tc_tiling_on_sc=False,
                                          internal_scratch_in_bytes=1 << 16)


def _deg_body(dst_h, zrow_h, deg_h, dstw, idxw, valw, zbuf, acc):
    sc = lax.axis_index("c")
    t = lax.axis_index("s")
    lo = sc * NHALF
    pltpu.sync_copy(zrow_h, zbuf)
    pltpu.sync_copy(zbuf, acc.at[pl.ds(t * RPT, RPT)])
    plsc.subcore_barrier()
    iota = lax.iota(jnp.int32, 16)

    def win(w, _):
        base = t * EPT + w * W
        pltpu.sync_copy(dst_h.at[pl.ds(base, W)], dstw)

        def grp(g, _):
            dv = dstw[pl.ds(g * 16, 16)]
            ev = base + g * 16 + iota
            m = (dv >= lo) & (dv < lo + NHALF)
            real = ev < ETOT
            idxw[pl.ds(g * 16, 16)] = jnp.where(m, dv - lo, ev % NHALF)
            valw[pl.ds(g * 16, 16)] = jnp.where(
                m & real, jnp.float32(1.0), jnp.float32(0.0))
            return 0

        lax.fori_loop(0, W // 16, grp, 0)
        pltpu.sync_copy(valw, acc.at[idxw], add=True)
        return 0

    lax.fori_loop(0, EPT // W, win, 0)
    plsc.subcore_barrier()
    pltpu.sync_copy(acc.at[pl.ds(t * RPT, RPT)], zbuf)
    pltpu.sync_copy(zbuf, deg_h.at[pl.ds(lo + t * RPT, RPT)])


_deg_kernel = functools.partial(
    pl.kernel,
    out_type=jax.ShapeDtypeStruct((NPAD,), jnp.float32),
    mesh=_mesh,
    compiler_params=_sc_params,
    scratch_types=[
        pltpu.VMEM((W,), jnp.int32),
        pltpu.VMEM((W,), jnp.int32),
        pltpu.VMEM((W,), jnp.float32),
        pltpu.VMEM((RPT,), jnp.float32),
        pltpu.VMEM_SHARED((NHALF,), jnp.float32),
    ],
)(_deg_body)


def _h0s_body(x_h, dinv_h, emb_h, out_h, xw, dw, embv, rows):
    sc = lax.axis_index("c")
    t = lax.axis_index("s")
    wid = t * 2 + sc
    g0 = wid * RPT
    pltpu.sync_copy(emb_h, embv)
    pltpu.sync_copy(x_h.at[pl.ds(g0, RPT)], xw)
    pltpu.sync_copy(dinv_h.at[pl.ds(g0, RPT)], dw)
    iota = lax.iota(jnp.int32, 16)
    half_rows = RPT // 2
    half_grps = half_rows // 16

    def half(hf, _):
        def grp(g, _):
            o = hf * half_rows + g * 16
            xv = xw[pl.ds(o, 16)]
            dv = dw[pl.ds(o, 16)]
            xbase = xv * D_EMB
            rbase = (g * 16 + iota) * D_EMB
            for c in range(D_EMB):
                col = plsc.load_gather(embv, [xbase + c])
                plsc.store_scatter(rows, [rbase + c], col * dv)
            return 0

        lax.fori_loop(0, half_grps, grp, 0)
        pltpu.sync_copy(
            rows, out_h.at[pl.ds((g0 + hf * half_rows) * D_EMB,
                                 half_rows * D_EMB)])
        return 0

    lax.fori_loop(0, 2, half, 0)


_h0s_kernel = functools.partial(
    pl.kernel,
    out_type=jax.ShapeDtypeStruct((NPAD * D_EMB,), jnp.float32),
    mesh=_mesh,
    compiler_params=_sc_params,
    scratch_types=[
        pltpu.VMEM((RPT,), jnp.int32),
        pltpu.VMEM((RPT,), jnp.float32),
        pltpu.VMEM((VOCAB * D_EMB,), jnp.float32),
        pltpu.VMEM((RPT // 2 * D_EMB,), jnp.float32),
    ],
)(_h0s_body)


def _agg_body(table_h, src_h, dst_h, zc_h, out_h,
              srcw, dstw, ssel, dsel, rows, zbuf, acc):
    sc = lax.axis_index("c")
    t = lax.axis_index("s")
    lo = sc * NHALF
    pltpu.sync_copy(zc_h, zbuf)
    for q in range(16):
        pltpu.sync_copy(zbuf, acc.at[pl.ds(t * RPT + q * 98, 98)])
    plsc.subcore_barrier()
    iota = lax.iota(jnp.int32, 16)

    def win(w, _):
        base = t * EPT + w * W
        pltpu.sync_copy(src_h.at[pl.ds(base, W)], srcw)
        pltpu.sync_copy(dst_h.at[pl.ds(base, W)], dstw)

        def grp(g, _):
            sv = srcw[pl.ds(g * 16, 16)]
            dv = dstw[pl.ds(g * 16, 16)]
            ev = base + g * 16 + iota
            m = (dv >= lo) & (dv < lo + NHALF)
            ssel[pl.ds(g * 16, 16)] = jnp.where(m, sv, N + (ev % ZROWS))
            dsel[pl.ds(g * 16, 16)] = jnp.where(m, dv - lo, ev % NHALF)
            return 0

        lax.fori_loop(0, W // 16, grp, 0)
        pltpu.sync_copy(table_h.at[ssel], rows)
        pltpu.sync_copy(rows, acc.at[dsel], add=True)
        return 0

    lax.fori_loop(0, EPT // W, win, 0)
    plsc.subcore_barrier()
    for q in range(16):
        pltpu.sync_copy(acc.at[pl.ds(t * RPT + q * 98, 98)], zbuf)
        pltpu.sync_copy(zbuf, out_h.at[pl.ds(lo + t * RPT + q * 98, 98)])


_agg_kernel = functools.partial(
    pl.kernel,
    out_type=jax.ShapeDtypeStruct((NPAD, D_EMB), jnp.float32),
    mesh=_mesh,
    compiler_params=_sc_params_untiled,
    scratch_types=[
        pltpu.VMEM((W,), jnp.int32),
        pltpu.VMEM((W,), jnp.int32),
        pltpu.VMEM((W,), jnp.int32),
        pltpu.VMEM((W,), jnp.int32),
        pltpu.VMEM((W, D_EMB), jnp.float32),
        pltpu.VMEM((98, D_EMB), jnp.float32),
        pltpu.VMEM_SHARED((NHALF, D_EMB), jnp.float32),
    ],
)(_agg_body)


def _dinv_body(deg_ref, out_ref):
    d = deg_ref[...]
    out_ref[...] = jnp.where(d > 0, lax.rsqrt(d), 0.0)


def _mm1_body(a_ref, dv_ref, w_ref, b_ref, oa_ref, ob_ref):
    a = a_ref[...] * dv_ref[...]
    h = jnp.dot(a, w_ref[...], preferred_element_type=jnp.float32) + b_ref[...]
    h = jnp.maximum(h, 0.0) * dv_ref[...]
    oa_ref[...] = h[:, :D_EMB]
    ob_ref[...] = h[:, D_EMB:]


def _mm2_body(a_ref, b_ref, dv_ref, bt_ref, w2_ref, bb2_ref, wl_ref, bl_ref,
              out_ref, psum, csum):
    i = pl.program_id(0)

    @pl.when(i == 0)
    def _():
        psum[...] = jnp.zeros_like(psum)
        csum[...] = jnp.zeros_like(csum)

    a = jnp.concatenate([a_ref[...], b_ref[...]], axis=1) * dv_ref[...]
    h = jnp.dot(a, w2_ref[...], preferred_element_type=jnp.float32) + bb2_ref[...]
    h = jnp.maximum(h, 0.0)
    bt = bt_ref[...]
    oh = (lax.broadcasted_iota(jnp.int32, (NG, BM), 0) == bt).astype(jnp.float32)
    psum[...] += jnp.dot(oh, h, preferred_element_type=jnp.float32)
    csum[...] += jnp.sum(oh, axis=1, keepdims=True)

    @pl.when(i == NBLK - 1)
    def _():
        pooled = psum[...] / jnp.maximum(csum[...], 1.0)
        out_ref[...] = jnp.dot(pooled, wl_ref[...],
                               preferred_element_type=jnp.float32) + bl_ref[...]


def kernel(x, edge_index, batch, emb, W1, b1, W2, b2, Wlin, blin):
    x = x.astype(jnp.int32)
    edge_index = edge_index.astype(jnp.int32)
    batch = batch.astype(jnp.int32)

    loop = jnp.arange(N, dtype=jnp.int32)
    padv = jnp.full((EPAD - ETOT,), NPAD - 1, jnp.int32)
    srcf = jnp.concatenate([edge_index[0], loop, padv])
    dstf = jnp.concatenate([edge_index[1], loop, padv])
    x_p = jnp.zeros((NPAD,), jnp.int32).at[:N].set(x)
    batch_row = jnp.full((1, NPAD), 1 << 20, jnp.int32).at[0, :N].set(batch)
    zrow = jnp.zeros((RPT,), jnp.float32)
    zc = jnp.zeros((98, D_EMB), jnp.float32)

    deg = _deg_kernel(dstf, zrow)
    dinv2d = pl.pallas_call(
        _dinv_body,
        out_shape=jax.ShapeDtypeStruct((NPAD // 128, 128), jnp.float32),
    )(deg.reshape(NPAD // 128, 128))
    dinv = dinv2d.reshape(NPAD)
    dinv_col = dinv.reshape(NPAD, 1)

    h0s = _h0s_kernel(x_p, dinv, emb.reshape(-1)).reshape(NPAD, D_EMB)

    agg1 = _agg_kernel(h0s, srcf, dstf, zc)

    h1s_a, h1s_b = pl.pallas_call(
        _mm1_body,
        grid=(NBLK,),
        in_specs=[
            pl.BlockSpec((BM, D_EMB), lambda i: (i, 0)),
            pl.BlockSpec((BM, 1), lambda i: (i, 0)),
            pl.BlockSpec((D_EMB, D_H), lambda i: (0, 0)),
            pl.BlockSpec((1, D_H), lambda i: (0, 0)),
        ],
        out_specs=[
            pl.BlockSpec((BM, D_EMB), lambda i: (i, 0)),
            pl.BlockSpec((BM, D_EMB), lambda i: (i, 0)),
        ],
        out_shape=[
            jax.ShapeDtypeStruct((NPAD, D_EMB), jnp.float32),
            jax.ShapeDtypeStruct((NPAD, D_EMB), jnp.float32),
        ],
    )(agg1, dinv_col, W1, b1.reshape(1, D_H))

    agg2a = _agg_kernel(h1s_a, srcf, dstf, zc)
    agg2b = _agg_kernel(h1s_b, srcf, dstf, zc)

    wlin_pad = jnp.zeros((D_H, 128), jnp.float32).at[:, :4].set(Wlin)
    blin_pad = jnp.zeros((1, 128), jnp.float32).at[0, :4].set(blin)
    out_pad = pl.pallas_call(
        _mm2_body,
        grid=(NBLK,),
        in_specs=[
            pl.BlockSpec((BM, D_EMB), lambda i: (i, 0)),
            pl.BlockSpec((BM, D_EMB), lambda i: (i, 0)),
            pl.BlockSpec((BM, 1), lambda i: (i, 0)),
            pl.BlockSpec((1, BM), lambda i: (0, i)),
            pl.BlockSpec((D_H, D_H), lambda i: (0, 0)),
            pl.BlockSpec((1, D_H), lambda i: (0, 0)),
            pl.BlockSpec((D_H, 128), lambda i: (0, 0)),
            pl.BlockSpec((1, 128), lambda i: (0, 0)),
        ],
        out_specs=pl.BlockSpec((NG, 128), lambda i: (0, 0)),
        out_shape=jax.ShapeDtypeStruct((NG, 128), jnp.float32),
        scratch_shapes=[
            pltpu.VMEM((NG, D_H), jnp.float32),
            pltpu.VMEM((NG, 1), jnp.float32),
        ],
    )(agg2a, agg2b, dinv_col, batch_row, W2, b2.reshape(1, D_H),
      wlin_pad, blin_pad)

    return out_pad[:, :4]


# speedup vs baseline: 9.8106x; 1.4128x over previous
import functools

import jax
import jax.numpy as jnp
from jax import lax
from jax.experimental import pallas as pl
from jax.experimental.pallas import tpu as pltpu
from jax.experimental.pallas import tpu_sc as plsc

N = 50000
NPAD = 50176
NHALF = NPAD // 2
RPT = NPAD // 32
ZROWS = NPAD - N
E = 800000
ETOT = E + N
EPAD = 851968
EPT = EPAD // 16
W = 128
D_EMB = 64
D_H = 128
VOCAB = 1024
NG = 256
BM = 512
NBLK = NPAD // BM

_mesh = plsc.VectorSubcoreMesh(core_axis_name="c", subcore_axis_name="s")
_sc_params = pltpu.CompilerParams(needs_layout_passes=False)
_sc_params_untiled = pltpu.CompilerParams(needs_layout_passes=False,
                                          use_tc_tiling_on_sc=False,
                                          internal_scratch_in_bytes=1 << 16)


def _deg_body(dst_h, zrow_h, deg_h, dstw, idxw, valw, zbuf, acc):
    sc = lax.axis_index("c")
    t = lax.axis_index("s")
    lo = sc * NHALF
    pltpu.sync_copy(zrow_h, zbuf)
    pltpu.sync_copy(zbuf, acc.at[pl.ds(t * RPT, RPT)])
    plsc.subcore_barrier()
    iota = lax.iota(jnp.int32, 16)

    def win(w, _):
        base = t * EPT + w * W
        pltpu.sync_copy(dst_h.at[pl.ds(base, W)], dstw)

        def grp(g, _):
            dv = dstw[pl.ds(g * 16, 16)]
            ev = base + g * 16 + iota
            m = (dv >= lo) & (dv < lo + NHALF)
            real = ev < ETOT
            idxw[pl.ds(g * 16, 16)] = jnp.where(m, dv - lo, ev % NHALF)
            valw[pl.ds(g * 16, 16)] = jnp.where(
                m & real, jnp.float32(1.0), jnp.float32(0.0))
            return 0

        lax.fori_loop(0, W // 16, grp, 0)
        pltpu.sync_copy(valw, acc.at[idxw], add=True)
        return 0

    lax.fori_loop(0, EPT // W, win, 0)
    plsc.subcore_barrier()
    pltpu.sync_copy(acc.at[pl.ds(t * RPT, RPT)], zbuf)
    pltpu.sync_copy(zbuf, deg_h.at[pl.ds(lo + t * RPT, RPT)])


_deg_kernel = functools.partial(
    pl.kernel,
    out_type=jax.ShapeDtypeStruct((NPAD,), jnp.float32),
    mesh=_mesh,
    compiler_params=_sc_params,
    scratch_types=[
        pltpu.VMEM((W,), jnp.int32),
        pltpu.VMEM((W,), jnp.int32),
        pltpu.VMEM((W,), jnp.float32),
        pltpu.VMEM((RPT,), jnp.float32),
        pltpu.VMEM_SHARED((NHALF,), jnp.float32),
    ],
)(_deg_body)


def _h0s_body(x_h, dinv_h, emb_h, out_h, xw, dw, embv, rows):
    sc = lax.axis_index("c")
    t = lax.axis_index("s")
    wid = t * 2 + sc
    g0 = wid * RPT
    pltpu.sync_copy(emb_h, embv)
    pltpu.sync_copy(x_h.at[pl.ds(g0, RPT)], xw)
    pltpu.sync_copy(dinv_h.at[pl.ds(g0, RPT)], dw)
    iota = lax.iota(jnp.int32, 16)
    half_rows = RPT // 2
    half_grps = half_rows // 16

    def half(hf, _):
        def grp(g, _):
            o = hf * half_rows + g * 16
            xv = xw[pl.ds(o, 16)]
            dv = dw[pl.ds(o, 16)]
            xbase = xv * D_EMB
            rbase = (g * 16 + iota) * D_EMB
            for c in range(D_EMB):
                col = plsc.load_gather(embv, [xbase + c])
                plsc.store_scatter(rows, [rbase + c], col * dv)
            return 0

        lax.fori_loop(0, half_grps, grp, 0)
        pltpu.sync_copy(
            rows, out_h.at[pl.ds((g0 + hf * half_rows) * D_EMB,
                                 half_rows * D_EMB)])
        return 0

    lax.fori_loop(0, 2, half, 0)


_h0s_kernel = functools.partial(
    pl.kernel,
    out_type=jax.ShapeDtypeStruct((NPAD * D_EMB,), jnp.float32),
    mesh=_mesh,
    compiler_params=_sc_params,
    scratch_types=[
        pltpu.VMEM((RPT,), jnp.int32),
        pltpu.VMEM((RPT,), jnp.float32),
        pltpu.VMEM((VOCAB * D_EMB,), jnp.float32),
        pltpu.VMEM((RPT // 2 * D_EMB,), jnp.float32),
    ],
)(_h0s_body)


CH = 2048
WPC = CH // W
NWIN = EPT // W


def _agg_body(table_h, src_h, dst_h, zc_h, out_h,
              srcc, dstc, ssel0, dsel0, ssel1, dsel1, rows0, rows1, zbuf, acc,
              gsem0, gsem1, ssem0, ssem1):
    sc = lax.axis_index("c")
    t = lax.axis_index("s")
    lo = sc * NHALF
    pltpu.sync_copy(zc_h, zbuf)
    for q in range(16):
        pltpu.sync_copy(zbuf, acc.at[pl.ds(t * RPT + q * 98, 98)])
    plsc.subcore_barrier()
    iota = lax.iota(jnp.int32, 16)

    def sel_compute(j, ssel, dsel):
        koff = (j % WPC) * W
        base = t * EPT + j * W

        def grp(g, _):
            sv = srcc[pl.ds(koff + g * 16, 16)]
            dv = dstc[pl.ds(koff + g * 16, 16)]
            ev = base + g * 16 + iota
            m = (dv >= lo) & (dv < lo + NHALF)
            ssel[pl.ds(g * 16, 16)] = jnp.where(m, sv, N + (ev & 127))
            dsel[pl.ds(g * 16, 16)] = jnp.where(m, dv - lo, ev & 16383)
            return 0

        lax.fori_loop(0, W // 16, grp, 0)

    def win(j, _):
        @pl.when(j % WPC == 0)
        def _():
            cbase = t * EPT + (j // WPC) * CH
            pltpu.sync_copy(src_h.at[pl.ds(cbase, CH)], srcc)
            pltpu.sync_copy(dst_h.at[pl.ds(cbase, CH)], dstc)

        def step(ssel_a, dsel_a, rows_a, gsem_a, ssem_a,
                 ssel_b, dsel_b, rows_b, gsem_b, ssem_b):
            @pl.when(j >= 2)
            def _():
                pltpu.make_async_copy(rows_a, acc.at[dsel_a], ssem_a).wait()

            sel_compute(j, ssel_a, dsel_a)
            pltpu.async_copy(table_h.at[ssel_a], rows_a, gsem_a)

            @pl.when(j >= 1)
            def _():
                pltpu.make_async_copy(table_h.at[ssel_b], rows_b, gsem_b).wait()
                pltpu.async_copy(rows_b, acc.at[dsel_b], ssem_b, add=True)

        @pl.when(j % 2 == 0)
        def _():
            step(ssel0, dsel0, rows0, gsem0, ssem0,
                 ssel1, dsel1, rows1, gsem1, ssem1)

        @pl.when(j % 2 == 1)
        def _():
            step(ssel1, dsel1, rows1, gsem1, ssem1,
                 ssel0, dsel0, rows0, gsem0, ssem0)

        return 0

    lax.fori_loop(0, NWIN, win, 0)
    pltpu.make_async_copy(table_h.at[ssel1], rows1, gsem1).wait()
    pltpu.async_copy(rows1, acc.at[dsel1], ssem1, add=True)
    pltpu.make_async_copy(rows0, acc.at[dsel0], ssem0).wait()
    pltpu.make_async_copy(rows1, acc.at[dsel1], ssem1).wait()
    plsc.subcore_barrier()
    for q in range(16):
        pltpu.sync_copy(acc.at[pl.ds(t * RPT + q * 98, 98)], zbuf)
        pltpu.sync_copy(zbuf, out_h.at[pl.ds(lo + t * RPT + q * 98, 98)])


_agg_kernel = functools.partial(
    pl.kernel,
    out_type=jax.ShapeDtypeStruct((NPAD, D_EMB), jnp.float32),
    mesh=_mesh,
    compiler_params=_sc_params_untiled,
    scratch_types=[
        pltpu.VMEM((CH,), jnp.int32),
        pltpu.VMEM((CH,), jnp.int32),
        pltpu.VMEM((W,), jnp.int32),
        pltpu.VMEM((W,), jnp.int32),
        pltpu.VMEM((W,), jnp.int32),
        pltpu.VMEM((W,), jnp.int32),
        pltpu.VMEM((W, D_EMB), jnp.float32),
        pltpu.VMEM((W, D_EMB), jnp.float32),
        pltpu.VMEM((98, D_EMB), jnp.float32),
        pltpu.VMEM_SHARED((NHALF, D_EMB), jnp.float32),
        pltpu.SemaphoreType.DMA,
        pltpu.SemaphoreType.DMA,
        pltpu.SemaphoreType.DMA,
        pltpu.SemaphoreType.DMA,
    ],
)(_agg_body)


def _dinv_body(deg_ref, out_ref):
    d = deg_ref[...]
    out_ref[...] = jnp.where(d > 0, lax.rsqrt(d), 0.0)


def _mm1_body(a_ref, dv_ref, w_ref, b_ref, oa_ref, ob_ref):
    a = a_ref[...] * dv_ref[...]
    h = jnp.dot(a, w_ref[...], preferred_element_type=jnp.float32, precision=lax.Precision.HIGHEST) + b_ref[...]
    h = jnp.maximum(h, 0.0) * dv_ref[...]
    oa_ref[...] = h[:, :D_EMB]
    ob_ref[...] = h[:, D_EMB:]


def _mm2_body(a_ref, b_ref, dv_ref, bt_ref, w2_ref, bb2_ref, wl_ref, bl_ref,
              out_ref, psum, csum):
    i = pl.program_id(0)

    @pl.when(i == 0)
    def _():
        psum[...] = jnp.zeros_like(psum)
        csum[...] = jnp.zeros_like(csum)

    a = jnp.concatenate([a_ref[...], b_ref[...]], axis=1) * dv_ref[...]
    h = jnp.dot(a, w2_ref[...], preferred_element_type=jnp.float32, precision=lax.Precision.HIGHEST) + bb2_ref[...]
    h = jnp.maximum(h, 0.0)
    bt = bt_ref[...]
    oh = (lax.broadcasted_iota(jnp.int32, (NG, BM), 0) == bt).astype(jnp.float32)
    psum[...] += jnp.dot(oh, h, preferred_element_type=jnp.float32, precision=lax.Precision.HIGHEST)
    csum[...] += jnp.sum(oh, axis=1, keepdims=True)

    @pl.when(i == NBLK - 1)
    def _():
        pooled = psum[...] / jnp.maximum(csum[...], 1.0)
        out_ref[...] = jnp.dot(pooled, wl_ref[...],
                               preferred_element_type=jnp.float32, precision=lax.Precision.HIGHEST) + bl_ref[...]


def kernel(x, edge_index, batch, emb, W1, b1, W2, b2, Wlin, blin):
    x = x.astype(jnp.int32)
    edge_index = edge_index.astype(jnp.int32)
    batch = batch.astype(jnp.int32)

    loop = jnp.arange(N, dtype=jnp.int32)
    padv = jnp.full((EPAD - ETOT,), NPAD - 1, jnp.int32)
    srcf = jnp.concatenate([edge_index[0], loop, padv])
    dstf = jnp.concatenate([edge_index[1], loop, padv])
    x_p = jnp.zeros((NPAD,), jnp.int32).at[:N].set(x)
    batch_row = jnp.full((1, NPAD), 1 << 20, jnp.int32).at[0, :N].set(batch)
    zrow = jnp.zeros((RPT,), jnp.float32)
    zc = jnp.zeros((98, D_EMB), jnp.float32)

    deg = _deg_kernel(dstf, zrow)
    dinv2d = pl.pallas_call(
        _dinv_body,
        out_shape=jax.ShapeDtypeStruct((NPAD // 128, 128), jnp.float32),
    )(deg.reshape(NPAD // 128, 128))
    dinv = dinv2d.reshape(NPAD)
    dinv_col = dinv.reshape(NPAD, 1)

    h0s = _h0s_kernel(x_p, dinv, emb.reshape(-1)).reshape(NPAD, D_EMB)

    agg1 = _agg_kernel(h0s, srcf, dstf, zc)

    h1s_a, h1s_b = pl.pallas_call(
        _mm1_body,
        grid=(NBLK,),
        in_specs=[
            pl.BlockSpec((BM, D_EMB), lambda i: (i, 0)),
            pl.BlockSpec((BM, 1), lambda i: (i, 0)),
            pl.BlockSpec((D_EMB, D_H), lambda i: (0, 0)),
            pl.BlockSpec((1, D_H), lambda i: (0, 0)),
        ],
        out_specs=[
            pl.BlockSpec((BM, D_EMB), lambda i: (i, 0)),
            pl.BlockSpec((BM, D_EMB), lambda i: (i, 0)),
        ],
        out_shape=[
            jax.ShapeDtypeStruct((NPAD, D_EMB), jnp.float32),
            jax.ShapeDtypeStruct((NPAD, D_EMB), jnp.float32),
        ],
    )(agg1, dinv_col, W1, b1.reshape(1, D_H))

    agg2a = _agg_kernel(h1s_a, srcf, dstf, zc)
    agg2b = _agg_kernel(h1s_b, srcf, dstf, zc)

    wlin_pad = jnp.zeros((D_H, 128), jnp.float32).at[:, :4].set(Wlin)
    blin_pad = jnp.zeros((1, 128), jnp.float32).at[0, :4].set(blin)
    out_pad = pl.pallas_call(
        _mm2_body,
        grid=(NBLK,),
        in_specs=[
            pl.BlockSpec((BM, D_EMB), lambda i: (i, 0)),
            pl.BlockSpec((BM, D_EMB), lambda i: (i, 0)),
            pl.BlockSpec((BM, 1), lambda i: (i, 0)),
            pl.BlockSpec((1, BM), lambda i: (0, i)),
            pl.BlockSpec((D_H, D_H), lambda i: (0, 0)),
            pl.BlockSpec((1, D_H), lambda i: (0, 0)),
            pl.BlockSpec((D_H, 128), lambda i: (0, 0)),
            pl.BlockSpec((1, 128), lambda i: (0, 0)),
        ],
        out_specs=pl.BlockSpec((NG, 128), lambda i: (0, 0)),
        out_shape=jax.ShapeDtypeStruct((NG, 128), jnp.float32),
        scratch_shapes=[
            pltpu.VMEM((NG, D_H), jnp.float32),
            pltpu.VMEM((NG, 1), jnp.float32),
        ],
    )(agg2a, agg2b, dinv_col, batch_row, W2, b2.reshape(1, D_H),
      wlin_pad, blin_pad)

    return out_pad[:, :4]


# speedup vs baseline: 20.9569x; 2.1361x over previous
import functools

import jax
import jax.numpy as jnp
from jax import lax
from jax.experimental import pallas as pl
from jax.experimental.pallas import tpu as pltpu
from jax.experimental.pallas import tpu_sc as plsc

N = 50000
NPAD = 50176
NHALF = NPAD // 2
RPT = NPAD // 32
ZROWS = NPAD - N
E = 800000
ETOT = E + N
EPAD = 851968
EPT = EPAD // 16
W = 128
CH = 2048
WPC = CH // W
D_EMB = 64
D_H = 128
VOCAB = 1024
NG = 256
BM = 512
NBLK = NPAD // BM

_mesh = plsc.VectorSubcoreMesh(core_axis_name="c", subcore_axis_name="s")
_sc_params = pltpu.CompilerParams(needs_layout_passes=False)
_sc_params_untiled = pltpu.CompilerParams(needs_layout_passes=False,
                                          use_tc_tiling_on_sc=False,
                                          internal_scratch_in_bytes=1 << 16)


def _part_body(src_h, dst_h, zrow_h, srcp_h, dstp_h, cnt_h, deg_h,
               srcc, dstc, degidx, degval, cbuf, zbuf, ssta, dsta, acc):
    sc = lax.axis_index("c")
    t = lax.axis_index("s")
    wid = t * 2 + sc
    lo = sc * NHALF
    pltpu.sync_copy(zrow_h, zbuf)
    pltpu.sync_copy(zbuf, acc.at[pl.ds(t * RPT, RPT)])
    plsc.subcore_barrier()
    iota = lax.iota(jnp.int32, 16)

    def chunk(cidx, off):
        cbase = t * EPT + cidx * CH
        pltpu.sync_copy(src_h.at[pl.ds(cbase, CH)], srcc)
        pltpu.sync_copy(dst_h.at[pl.ds(cbase, CH)], dstc)

        def win(w, off):
            def grp(g, off):
                o = w * W + g * 16
                sv = srcc[pl.ds(o, 16)]
                dv = dstc[pl.ds(o, 16)]
                ev = cbase + o + iota
                m = (dv >= lo) & (dv < lo + NHALF) & (ev < ETOT)
                dloc = dv - lo
                plsc.store_compressed(ssta.at[pl.ds(off, 16)], sv, mask=m)
                plsc.store_compressed(dsta.at[pl.ds(off, 16)], dloc, mask=m)
                degidx[pl.ds(g * 16, 16)] = jnp.where(m, dloc, ev & 16383)
                degval[pl.ds(g * 16, 16)] = jnp.where(
                    m, jnp.float32(1.0), jnp.float32(0.0))
                return off + jnp.sum(m.astype(jnp.int32))

            off = lax.fori_loop(0, W // 16, grp, off)
            pltpu.sync_copy(degval, acc.at[degidx], add=True)
            return off

        return lax.fori_loop(0, WPC, win, off)

    off = lax.fori_loop(0, EPT // CH, chunk, jnp.int32(0))

    for k in range(8):
        ssta[pl.ds(off + k * 16, 16)] = N + iota
        dsta[pl.ds(off + k * 16, 16)] = iota
    cnt128 = ((off + 127) // 128) * 128
    cbuf[...] = jnp.full((16,), 0, jnp.int32) + cnt128
    pltpu.sync_copy(cbuf, cnt_h.at[wid])
    obase = sc * EPAD + t * EPT

    def flush(f, _):
        pltpu.sync_copy(ssta.at[pl.ds(f * CH, CH)],
                        srcp_h.at[pl.ds(obase + f * CH, CH)])
        pltpu.sync_copy(dsta.at[pl.ds(f * CH, CH)],
                        dstp_h.at[pl.ds(obase + f * CH, CH)])
        return 0

    lax.fori_loop(0, (cnt128 + CH - 1) // CH, flush, 0)
    plsc.subcore_barrier()
    pltpu.sync_copy(acc.at[pl.ds(t * RPT, RPT)], zbuf)
    pltpu.sync_copy(zbuf, deg_h.at[pl.ds(lo + t * RPT, RPT)])


_part_kernel = functools.partial(
    pl.kernel,
    out_type=[
        jax.ShapeDtypeStruct((2 * EPAD,), jnp.int32),
        jax.ShapeDtypeStruct((2 * EPAD,), jnp.int32),
        jax.ShapeDtypeStruct((32, 16), jnp.int32),
        jax.ShapeDtypeStruct((NPAD,), jnp.float32),
    ],
    mesh=_mesh,
    compiler_params=_sc_params,
    scratch_types=[
        pltpu.VMEM((CH,), jnp.int32),
        pltpu.VMEM((CH,), jnp.int32),
        pltpu.VMEM((W,), jnp.int32),
        pltpu.VMEM((W,), jnp.float32),
        pltpu.VMEM((16,), jnp.int32),
        pltpu.VMEM((RPT,), jnp.float32),
        pltpu.VMEM((EPT + 128,), jnp.int32),
        pltpu.VMEM((EPT + 128,), jnp.int32),
        pltpu.VMEM_SHARED((NHALF,), jnp.float32),
    ],
)(_part_body)


def _h0s_body(x_h, dinv_h, emb_h, out_h, xw, dw, embv, rows):
    sc = lax.axis_index("c")
    t = lax.axis_index("s")
    wid = t * 2 + sc
    g0 = wid * RPT
    pltpu.sync_copy(emb_h, embv)
    pltpu.sync_copy(x_h.at[pl.ds(g0, RPT)], xw)
    pltpu.sync_copy(dinv_h.at[pl.ds(g0, RPT)], dw)
    iota = lax.iota(jnp.int32, 16)
    half_rows = RPT // 2
    half_grps = half_rows // 16

    def half(hf, _):
        def grp(g, _):
            o = hf * half_rows + g * 16
            xv = xw[pl.ds(o, 16)]
            dv = dw[pl.ds(o, 16)]
            xbase = xv * D_EMB
            rbase = (g * 16 + iota) * D_EMB
            for c in range(D_EMB):
                col = plsc.load_gather(embv, [xbase + c])
                plsc.store_scatter(rows, [rbase + c], col * dv)
            return 0

        lax.fori_loop(0, half_grps, grp, 0)
        pltpu.sync_copy(
            rows, out_h.at[pl.ds((g0 + hf * half_rows) * D_EMB,
                                 half_rows * D_EMB)])
        return 0

    lax.fori_loop(0, 2, half, 0)


_h0s_kernel = functools.partial(
    pl.kernel,
    out_type=jax.ShapeDtypeStruct((NPAD * D_EMB,), jnp.float32),
    mesh=_mesh,
    compiler_params=_sc_params,
    scratch_types=[
        pltpu.VMEM((RPT,), jnp.int32),
        pltpu.VMEM((RPT,), jnp.float32),
        pltpu.VMEM((VOCAB * D_EMB,), jnp.float32),
        pltpu.VMEM((RPT // 2 * D_EMB,), jnp.float32),
    ],
)(_h0s_body)


def _agg_body(table_h, srcp_h, dstp_h, cnt_h, zc_h, out_h,
              srcc, dstc, ssel0, dsel0, ssel1, dsel1, rows0, rows1, zbuf,
              cbuf, acc, gsem0, gsem1, ssem0, ssem1):
    sc = lax.axis_index("c")
    t = lax.axis_index("s")
    wid = t * 2 + sc
    lo = sc * NHALF
    pltpu.sync_copy(zc_h, zbuf)
    for q in range(16):
        pltpu.sync_copy(zbuf, acc.at[pl.ds(t * RPT + q * 98, 98)])
    pltpu.sync_copy(cnt_h.at[wid], cbuf)
    nw = cbuf[pl.ds(0, 16)][0] // W
    ebase = sc * EPAD + t * EPT
    plsc.subcore_barrier()

    def sel_compute(j, ssel, dsel):
        koff = (j % WPC) * W

        def grp(g, _):
            ssel[pl.ds(g * 16, 16)] = srcc[pl.ds(koff + g * 16, 16)]
            dsel[pl.ds(g * 16, 16)] = dstc[pl.ds(koff + g * 16, 16)]
            return 0

        lax.fori_loop(0, W // 16, grp, 0)

    def win(j, _):
        @pl.when(j % WPC == 0)
        def _():
            cbase = ebase + (j // WPC) * CH
            pltpu.sync_copy(srcp_h.at[pl.ds(cbase, CH)], srcc)
            pltpu.sync_copy(dstp_h.at[pl.ds(cbase, CH)], dstc)

        def step(ssel_a, dsel_a, rows_a, gsem_a, ssem_a,
                 ssel_b, dsel_b, rows_b, gsem_b, ssem_b):
            @pl.when(j >= 2)
            def _():
                pltpu.make_async_copy(rows_a, acc.at[dsel_a], ssem_a).wait()

            sel_compute(j, ssel_a, dsel_a)
            pltpu.async_copy(table_h.at[ssel_a], rows_a, gsem_a)

            @pl.when(j >= 1)
            def _():
                pltpu.make_async_copy(table_h.at[ssel_b], rows_b, gsem_b).wait()
                pltpu.async_copy(rows_b, acc.at[dsel_b], ssem_b, add=True)

        @pl.when(j % 2 == 0)
        def _():
            step(ssel0, dsel0, rows0, gsem0, ssem0,
                 ssel1, dsel1, rows1, gsem1, ssem1)

        @pl.when(j % 2 == 1)
        def _():
            step(ssel1, dsel1, rows1, gsem1, ssem1,
                 ssel0, dsel0, rows0, gsem0, ssem0)

        return 0

    lax.fori_loop(0, nw, win, 0)
    blast = (nw - 1) % 2

    @pl.when((nw >= 1) & (blast == 0))
    def _():
        pltpu.make_async_copy(table_h.at[ssel0], rows0, gsem0).wait()
        pltpu.async_copy(rows0, acc.at[dsel0], ssem0, add=True)

    @pl.when((nw >= 1) & (blast == 1))
    def _():
        pltpu.make_async_copy(table_h.at[ssel1], rows1, gsem1).wait()
        pltpu.async_copy(rows1, acc.at[dsel1], ssem1, add=True)

    @pl.when((nw >= 2) & (blast == 0))
    def _():
        pltpu.make_async_copy(rows1, acc.at[dsel1], ssem1).wait()

    @pl.when((nw >= 2) & (blast == 1))
    def _():
        pltpu.make_async_copy(rows0, acc.at[dsel0], ssem0).wait()

    @pl.when((nw >= 1) & (blast == 0))
    def _():
        pltpu.make_async_copy(rows0, acc.at[dsel0], ssem0).wait()

    @pl.when((nw >= 1) & (blast == 1))
    def _():
        pltpu.make_async_copy(rows1, acc.at[dsel1], ssem1).wait()

    plsc.subcore_barrier()
    for q in range(16):
        pltpu.sync_copy(acc.at[pl.ds(t * RPT + q * 98, 98)], zbuf)
        pltpu.sync_copy(zbuf, out_h.at[pl.ds(lo + t * RPT + q * 98, 98)])


_agg_kernel = functools.partial(
    pl.kernel,
    out_type=jax.ShapeDtypeStruct((NPAD, D_EMB), jnp.float32),
    mesh=_mesh,
    compiler_params=_sc_params_untiled,
    scratch_types=[
        pltpu.VMEM((CH,), jnp.int32),
        pltpu.VMEM((CH,), jnp.int32),
        pltpu.VMEM((W,), jnp.int32),
        pltpu.VMEM((W,), jnp.int32),
        pltpu.VMEM((W,), jnp.int32),
        pltpu.VMEM((W,), jnp.int32),
        pltpu.VMEM((W, D_EMB), jnp.float32),
        pltpu.VMEM((W, D_EMB), jnp.float32),
        pltpu.VMEM((98, D_EMB), jnp.float32),
        pltpu.VMEM((16,), jnp.int32),
        pltpu.VMEM_SHARED((NHALF, D_EMB), jnp.float32),
        pltpu.SemaphoreType.DMA,
        pltpu.SemaphoreType.DMA,
        pltpu.SemaphoreType.DMA,
        pltpu.SemaphoreType.DMA,
    ],
)(_agg_body)


def _dinv_body(deg_ref, out_ref):
    d = deg_ref[...]
    out_ref[...] = jnp.where(d > 0, lax.rsqrt(d), 0.0)


def _mm1_body(a_ref, dv_ref, w_ref, b_ref, oa_ref, ob_ref):
    a = a_ref[...] * dv_ref[...]
    h = jnp.dot(a, w_ref[...], preferred_element_type=jnp.float32, precision=lax.Precision.HIGHEST) + b_ref[...]
    h = jnp.maximum(h, 0.0) * dv_ref[...]
    oa_ref[...] = h[:, :D_EMB]
    ob_ref[...] = h[:, D_EMB:]


def _mm2_body(a_ref, b_ref, dv_ref, bt_ref, w2_ref, bb2_ref, wl_ref, bl_ref,
              out_ref, psum, csum):
    i = pl.program_id(0)

    @pl.when(i == 0)
    def _():
        psum[...] = jnp.zeros_like(psum)
        csum[...] = jnp.zeros_like(csum)

    a = jnp.concatenate([a_ref[...], b_ref[...]], axis=1) * dv_ref[...]
    h = jnp.dot(a, w2_ref[...], preferred_element_type=jnp.float32, precision=lax.Precision.HIGHEST) + bb2_ref[...]
    h = jnp.maximum(h, 0.0)
    bt = bt_ref[...]
    oh = (lax.broadcasted_iota(jnp.int32, (NG, BM), 0) == bt).astype(jnp.float32)
    psum[...] += jnp.dot(oh, h, preferred_element_type=jnp.float32, precision=lax.Precision.HIGHEST)
    csum[...] += jnp.sum(oh, axis=1, keepdims=True)

    @pl.when(i == NBLK - 1)
    def _():
        pooled = psum[...] / jnp.maximum(csum[...], 1.0)
        out_ref[...] = jnp.dot(pooled, wl_ref[...],
                               preferred_element_type=jnp.float32, precision=lax.Precision.HIGHEST) + bl_ref[...]


def kernel(x, edge_index, batch, emb, W1, b1, W2, b2, Wlin, blin):
    x = x.astype(jnp.int32)
    edge_index = edge_index.astype(jnp.int32)
    batch = batch.astype(jnp.int32)

    loop = jnp.arange(N, dtype=jnp.int32)
    padv = jnp.full((EPAD - ETOT,), NPAD - 1, jnp.int32)
    srcf = jnp.concatenate([edge_index[0], loop, padv])
    dstf = jnp.concatenate([edge_index[1], loop, padv])
    x_p = jnp.zeros((NPAD,), jnp.int32).at[:N].set(x)
    batch_row = jnp.full((1, NPAD), 1 << 20, jnp.int32).at[0, :N].set(batch)
    zrow = jnp.zeros((RPT,), jnp.float32)
    zc = jnp.zeros((98, D_EMB), jnp.float32)

    srcp, dstp, cnt, deg = _part_kernel(srcf, dstf, zrow)
    dinv2d = pl.pallas_call(
        _dinv_body,
        out_shape=jax.ShapeDtypeStruct((NPAD // 128, 128), jnp.float32),
    )(deg.reshape(NPAD // 128, 128))
    dinv = dinv2d.reshape(NPAD)
    dinv_col = dinv.reshape(NPAD, 1)

    h0s = _h0s_kernel(x_p, dinv, emb.reshape(-1)).reshape(NPAD, D_EMB)

    agg1 = _agg_kernel(h0s, srcp, dstp, cnt, zc)

    h1s_a, h1s_b = pl.pallas_call(
        _mm1_body,
        grid=(NBLK,),
        in_specs=[
            pl.BlockSpec((BM, D_EMB), lambda i: (i, 0)),
            pl.BlockSpec((BM, 1), lambda i: (i, 0)),
            pl.BlockSpec((D_EMB, D_H), lambda i: (0, 0)),
            pl.BlockSpec((1, D_H), lambda i: (0, 0)),
        ],
        out_specs=[
            pl.BlockSpec((BM, D_EMB), lambda i: (i, 0)),
            pl.BlockSpec((BM, D_EMB), lambda i: (i, 0)),
        ],
        out_shape=[
            jax.ShapeDtypeStruct((NPAD, D_EMB), jnp.float32),
            jax.ShapeDtypeStruct((NPAD, D_EMB), jnp.float32),
        ],
    )(agg1, dinv_col, W1, b1.reshape(1, D_H))

    agg2a = _agg_kernel(h1s_a, srcp, dstp, cnt, zc)
    agg2b = _agg_kernel(h1s_b, srcp, dstp, cnt, zc)

    wlin_pad = jnp.zeros((D_H, 128), jnp.float32).at[:, :4].set(Wlin)
    blin_pad = jnp.zeros((1, 128), jnp.float32).at[0, :4].set(blin)
    out_pad = pl.pallas_call(
        _mm2_body,
        grid=(NBLK,),
        in_specs=[
            pl.BlockSpec((BM, D_EMB), lambda i: (i, 0)),
            pl.BlockSpec((BM, D_EMB), lambda i: (i, 0)),
            pl.BlockSpec((BM, 1), lambda i: (i, 0)),
            pl.BlockSpec((1, BM), lambda i: (0, i)),
            pl.BlockSpec((D_H, D_H), lambda i: (0, 0)),
            pl.BlockSpec((1, D_H), lambda i: (0, 0)),
            pl.BlockSpec((D_H, 128), lambda i: (0, 0)),
            pl.BlockSpec((1, 128), lambda i: (0, 0)),
        ],
        out_specs=pl.BlockSpec((NG, 128), lambda i: (0, 0)),
        out_shape=jax.ShapeDtypeStruct((NG, 128), jnp.float32),
        scratch_shapes=[
            pltpu.VMEM((NG, D_H), jnp.float32),
            pltpu.VMEM((NG, 1), jnp.float32),
        ],
    )(agg2a, agg2b, dinv_col, batch_row, W2, b2.reshape(1, D_H),
      wlin_pad, blin_pad)

    return out_pad[:, :4]


# speedup vs baseline: 22.4844x; 1.0729x over previous
import functools

import jax
import jax.numpy as jnp
from jax import lax
from jax.experimental import pallas as pl
from jax.experimental.pallas import tpu as pltpu
from jax.experimental.pallas import tpu_sc as plsc

N = 50000
NPAD = 50176
NHALF = NPAD // 2
RPT = NPAD // 32
ZROWS = NPAD - N
E = 800000
EPAD = 819200
EPT = EPAD // 16
W = 128
CH = 2048
WPC = CH // W
D_EMB = 64
D_H = 128
VOCAB = 1024
NG = 256
BM = 512
NBLK = NPAD // BM

_mesh = plsc.VectorSubcoreMesh(core_axis_name="c", subcore_axis_name="s")
_sc_params = pltpu.CompilerParams(needs_layout_passes=False)
_sc_params_untiled = pltpu.CompilerParams(needs_layout_passes=False,
                                          use_tc_tiling_on_sc=False,
                                          internal_scratch_in_bytes=1 << 16)


def _part_body(src_h, dst_h, zrow_h, srcp_h, dstp_h, cnt_h, deg_h,
               srcc, dstc, degidx, degval, cbuf, zbuf, ssta, dsta, acc):
    sc = lax.axis_index("c")
    t = lax.axis_index("s")
    wid = t * 2 + sc
    lo = sc * NHALF
    pltpu.sync_copy(zrow_h, zbuf)
    pltpu.sync_copy(zbuf, acc.at[pl.ds(t * RPT, RPT)])
    plsc.subcore_barrier()
    iota = lax.iota(jnp.int32, 16)

    def chunk(cidx, off):
        cbase = t * EPT + cidx * CH
        pltpu.sync_copy(src_h.at[pl.ds(cbase, CH)], srcc)
        pltpu.sync_copy(dst_h.at[pl.ds(cbase, CH)], dstc)

        def win(w, off):
            def grp(g, off):
                o = w * W + g * 16
                sv = srcc[pl.ds(o, 16)]
                dv = dstc[pl.ds(o, 16)]
                ev = cbase + o + iota
                m = (dv >= lo) & (dv < lo + NHALF) & (ev < E)
                dloc = dv - lo
                plsc.store_compressed(ssta.at[pl.ds(off, 16)], sv, mask=m)
                plsc.store_compressed(dsta.at[pl.ds(off, 16)], dloc, mask=m)
                degidx[pl.ds(g * 16, 16)] = jnp.where(m, dloc, ev & 16383)
                degval[pl.ds(g * 16, 16)] = jnp.where(
                    m, jnp.float32(1.0), jnp.float32(0.0))
                return off + jnp.sum(m.astype(jnp.int32))

            off = lax.fori_loop(0, W // 16, grp, off)
            pltpu.sync_copy(degval, acc.at[degidx], add=True)
            return off

        return lax.fori_loop(0, WPC, win, off)

    off = lax.fori_loop(0, EPT // CH, chunk, jnp.int32(0))

    for k in range(8):
        ssta[pl.ds(off + k * 16, 16)] = N + iota
        dsta[pl.ds(off + k * 16, 16)] = iota
    cnt128 = ((off + 127) // 128) * 128
    cbuf[...] = jnp.full((16,), 0, jnp.int32) + cnt128
    pltpu.sync_copy(cbuf, cnt_h.at[wid])
    obase = sc * EPAD + t * EPT

    def flush(f, _):
        pltpu.sync_copy(ssta.at[pl.ds(f * CH, CH)],
                        srcp_h.at[pl.ds(obase + f * CH, CH)])
        pltpu.sync_copy(dsta.at[pl.ds(f * CH, CH)],
                        dstp_h.at[pl.ds(obase + f * CH, CH)])
        return 0

    lax.fori_loop(0, (cnt128 + CH - 1) // CH, flush, 0)
    plsc.subcore_barrier()
    pltpu.sync_copy(acc.at[pl.ds(t * RPT, RPT)], zbuf)
    pltpu.sync_copy(zbuf, deg_h.at[pl.ds(lo + t * RPT, RPT)])


_part_kernel = functools.partial(
    pl.kernel,
    out_type=[
        jax.ShapeDtypeStruct((2 * EPAD,), jnp.int32),
        jax.ShapeDtypeStruct((2 * EPAD,), jnp.int32),
        jax.ShapeDtypeStruct((32, 16), jnp.int32),
        jax.ShapeDtypeStruct((NPAD,), jnp.float32),
    ],
    mesh=_mesh,
    compiler_params=_sc_params,
    scratch_types=[
        pltpu.VMEM((CH,), jnp.int32),
        pltpu.VMEM((CH,), jnp.int32),
        pltpu.VMEM((W,), jnp.int32),
        pltpu.VMEM((W,), jnp.float32),
        pltpu.VMEM((16,), jnp.int32),
        pltpu.VMEM((RPT,), jnp.float32),
        pltpu.VMEM((EPT + 128,), jnp.int32),
        pltpu.VMEM((EPT + 128,), jnp.int32),
        pltpu.VMEM_SHARED((NHALF,), jnp.float32),
    ],
)(_part_body)


def _h0s_body(x_h, dinv_h, emb_h, out_h, xw, dw, embv, rows):
    sc = lax.axis_index("c")
    t = lax.axis_index("s")
    wid = t * 2 + sc
    g0 = wid * RPT
    pltpu.sync_copy(emb_h, embv)
    pltpu.sync_copy(x_h.at[pl.ds(g0, RPT)], xw)
    pltpu.sync_copy(dinv_h.at[pl.ds(g0, RPT)], dw)
    iota = lax.iota(jnp.int32, 16)
    half_rows = RPT // 2
    half_grps = half_rows // 16

    def half(hf, _):
        def grp(g, _):
            o = hf * half_rows + g * 16
            xv = xw[pl.ds(o, 16)]
            dv = dw[pl.ds(o, 16)]
            xbase = xv * D_EMB
            rbase = (g * 16 + iota) * D_EMB
            for c in range(D_EMB):
                col = plsc.load_gather(embv, [xbase + c])
                plsc.store_scatter(rows, [rbase + c], col * dv)
            return 0

        lax.fori_loop(0, half_grps, grp, 0)
        pltpu.sync_copy(
            rows, out_h.at[pl.ds((g0 + hf * half_rows) * D_EMB,
                                 half_rows * D_EMB)])
        return 0

    lax.fori_loop(0, 2, half, 0)


_h0s_kernel = functools.partial(
    pl.kernel,
    out_type=jax.ShapeDtypeStruct((NPAD * D_EMB,), jnp.float32),
    mesh=_mesh,
    compiler_params=_sc_params,
    scratch_types=[
        pltpu.VMEM((RPT,), jnp.int32),
        pltpu.VMEM((RPT,), jnp.float32),
        pltpu.VMEM((VOCAB * D_EMB,), jnp.float32),
        pltpu.VMEM((RPT // 2 * D_EMB,), jnp.float32),
    ],
)(_h0s_body)


def _agg_body(table_h, srcp_h, dstp_h, cnt_h, zc_h, out_h,
              srcc, dstc, ssel0, dsel0, ssel1, dsel1, rows0, rows1, zbuf,
              cbuf, acc, gsem0, gsem1, ssem0, ssem1):
    sc = lax.axis_index("c")
    t = lax.axis_index("s")
    wid = t * 2 + sc
    lo = sc * NHALF
    pltpu.sync_copy(zc_h, zbuf)
    for q in range(16):
        pltpu.sync_copy(zbuf, acc.at[pl.ds(t * RPT + q * 98, 98)])
    pltpu.sync_copy(cnt_h.at[wid], cbuf)
    nw = cbuf[pl.ds(0, 16)][0] // W
    ebase = sc * EPAD + t * EPT
    plsc.subcore_barrier()

    def sel_compute(j, ssel, dsel):
        koff = (j % WPC) * W

        def grp(g, _):
            ssel[pl.ds(g * 16, 16)] = srcc[pl.ds(koff + g * 16, 16)]
            dsel[pl.ds(g * 16, 16)] = dstc[pl.ds(koff + g * 16, 16)]
            return 0

        lax.fori_loop(0, W // 16, grp, 0)

    def win(j, _):
        @pl.when(j % WPC == 0)
        def _():
            cbase = ebase + (j // WPC) * CH
            pltpu.sync_copy(srcp_h.at[pl.ds(cbase, CH)], srcc)
            pltpu.sync_copy(dstp_h.at[pl.ds(cbase, CH)], dstc)

        def step(ssel_a, dsel_a, rows_a, gsem_a, ssem_a,
                 ssel_b, dsel_b, rows_b, gsem_b, ssem_b):
            @pl.when(j >= 2)
            def _():
                pltpu.make_async_copy(rows_a, acc.at[dsel_a], ssem_a).wait()

            sel_compute(j, ssel_a, dsel_a)
            pltpu.async_copy(table_h.at[ssel_a], rows_a, gsem_a)

            @pl.when(j >= 1)
            def _():
                pltpu.make_async_copy(table_h.at[ssel_b], rows_b, gsem_b).wait()
                pltpu.async_copy(rows_b, acc.at[dsel_b], ssem_b, add=True)

        @pl.when(j % 2 == 0)
        def _():
            step(ssel0, dsel0, rows0, gsem0, ssem0,
                 ssel1, dsel1, rows1, gsem1, ssem1)

        @pl.when(j % 2 == 1)
        def _():
            step(ssel1, dsel1, rows1, gsem1, ssem1,
                 ssel0, dsel0, rows0, gsem0, ssem0)

        return 0

    lax.fori_loop(0, nw, win, 0)
    blast = (nw - 1) % 2

    @pl.when((nw >= 1) & (blast == 0))
    def _():
        pltpu.make_async_copy(table_h.at[ssel0], rows0, gsem0).wait()
        pltpu.async_copy(rows0, acc.at[dsel0], ssem0, add=True)

    @pl.when((nw >= 1) & (blast == 1))
    def _():
        pltpu.make_async_copy(table_h.at[ssel1], rows1, gsem1).wait()
        pltpu.async_copy(rows1, acc.at[dsel1], ssem1, add=True)

    @pl.when((nw >= 2) & (blast == 0))
    def _():
        pltpu.make_async_copy(rows1, acc.at[dsel1], ssem1).wait()

    @pl.when((nw >= 2) & (blast == 1))
    def _():
        pltpu.make_async_copy(rows0, acc.at[dsel0], ssem0).wait()

    @pl.when((nw >= 1) & (blast == 0))
    def _():
        pltpu.make_async_copy(rows0, acc.at[dsel0], ssem0).wait()

    @pl.when((nw >= 1) & (blast == 1))
    def _():
        pltpu.make_async_copy(rows1, acc.at[dsel1], ssem1).wait()

    plsc.subcore_barrier()
    for q in range(16):
        pltpu.sync_copy(acc.at[pl.ds(t * RPT + q * 98, 98)], zbuf)
        pltpu.sync_copy(zbuf, out_h.at[pl.ds(lo + t * RPT + q * 98, 98)])


_agg_kernel = functools.partial(
    pl.kernel,
    out_type=jax.ShapeDtypeStruct((NPAD, D_EMB), jnp.float32),
    mesh=_mesh,
    compiler_params=_sc_params_untiled,
    scratch_types=[
        pltpu.VMEM((CH,), jnp.int32),
        pltpu.VMEM((CH,), jnp.int32),
        pltpu.VMEM((W,), jnp.int32),
        pltpu.VMEM((W,), jnp.int32),
        pltpu.VMEM((W,), jnp.int32),
        pltpu.VMEM((W,), jnp.int32),
        pltpu.VMEM((W, D_EMB), jnp.float32),
        pltpu.VMEM((W, D_EMB), jnp.float32),
        pltpu.VMEM((98, D_EMB), jnp.float32),
        pltpu.VMEM((16,), jnp.int32),
        pltpu.VMEM_SHARED((NHALF, D_EMB), jnp.float32),
        pltpu.SemaphoreType.DMA,
        pltpu.SemaphoreType.DMA,
        pltpu.SemaphoreType.DMA,
        pltpu.SemaphoreType.DMA,
    ],
)(_agg_body)


def _dinv_body(deg_ref, out_ref):
    d = deg_ref[...]
    i0 = lax.broadcasted_iota(jnp.int32, d.shape, 0)
    i1 = lax.broadcasted_iota(jnp.int32, d.shape, 1)
    row = i0 * 128 + i1
    out_ref[...] = jnp.where(row < N, lax.rsqrt(d + 1.0), 0.0)


def _mm1_body(a_ref, hs_ref, dv_ref, w_ref, b_ref, oa_ref, ob_ref):
    a = (a_ref[...] + hs_ref[...]) * dv_ref[...]
    h = jnp.dot(a, w_ref[...], preferred_element_type=jnp.float32, precision=lax.Precision.HIGHEST) + b_ref[...]
    h = jnp.maximum(h, 0.0) * dv_ref[...]
    oa_ref[...] = h[:, :D_EMB]
    ob_ref[...] = h[:, D_EMB:]


def _mm2_body(a_ref, b_ref, ha_ref, hb_ref, dv_ref, bt_ref, w2_ref, bb2_ref,
              wl_ref, bl_ref, out_ref, psum, csum):
    i = pl.program_id(0)

    @pl.when(i == 0)
    def _():
        psum[...] = jnp.zeros_like(psum)
        csum[...] = jnp.zeros_like(csum)

    a = (jnp.concatenate([a_ref[...] + ha_ref[...],
                          b_ref[...] + hb_ref[...]], axis=1)) * dv_ref[...]
    h = jnp.dot(a, w2_ref[...], preferred_element_type=jnp.float32, precision=lax.Precision.HIGHEST) + bb2_ref[...]
    h = jnp.maximum(h, 0.0)
    bt = bt_ref[...]
    oh = (lax.broadcasted_iota(jnp.int32, (NG, BM), 0) == bt).astype(jnp.float32)
    psum[...] += jnp.dot(oh, h, preferred_element_type=jnp.float32, precision=lax.Precision.HIGHEST)
    csum[...] += jnp.sum(oh, axis=1, keepdims=True)

    @pl.when(i == NBLK - 1)
    def _():
        pooled = psum[...] / jnp.maximum(csum[...], 1.0)
        out_ref[...] = jnp.dot(pooled, wl_ref[...],
                               preferred_element_type=jnp.float32, precision=lax.Precision.HIGHEST) + bl_ref[...]


def kernel(x, edge_index, batch, emb, W1, b1, W2, b2, Wlin, blin):
    x = x.astype(jnp.int32)
    edge_index = edge_index.astype(jnp.int32)
    batch = batch.astype(jnp.int32)

    ei = jnp.full((2, EPAD), NPAD - 1, jnp.int32).at[:, :E].set(edge_index)
    srcf = ei[0]
    dstf = ei[1]
    x_p = jnp.zeros((NPAD,), jnp.int32).at[:N].set(x)
    batch_row = jnp.full((1, NPAD), 1 << 20, jnp.int32).at[0, :N].set(batch)
    zrow = jnp.zeros((RPT,), jnp.float32)
    zc = jnp.zeros((98, D_EMB), jnp.float32)

    srcp, dstp, cnt, deg = _part_kernel(srcf, dstf, zrow)
    dinv2d = pl.pallas_call(
        _dinv_body,
        out_shape=jax.ShapeDtypeStruct((NPAD // 128, 128), jnp.float32),
    )(deg.reshape(NPAD // 128, 128))
    dinv = dinv2d.reshape(NPAD)
    dinv_col = dinv.reshape(NPAD, 1)

    h0s = _h0s_kernel(x_p, dinv, emb.reshape(-1)).reshape(NPAD, D_EMB)

    agg1 = _agg_kernel(h0s, srcp, dstp, cnt, zc)

    h1s_a, h1s_b = pl.pallas_call(
        _mm1_body,
        grid=(NBLK,),
        in_specs=[
            pl.BlockSpec((BM, D_EMB), lambda i: (i, 0)),
            pl.BlockSpec((BM, D_EMB), lambda i: (i, 0)),
            pl.BlockSpec((BM, 1), lambda i: (i, 0)),
            pl.BlockSpec((D_EMB, D_H), lambda i: (0, 0)),
            pl.BlockSpec((1, D_H), lambda i: (0, 0)),
        ],
        out_specs=[
            pl.BlockSpec((BM, D_EMB), lambda i: (i, 0)),
            pl.BlockSpec((BM, D_EMB), lambda i: (i, 0)),
        ],
        out_shape=[
            jax.ShapeDtypeStruct((NPAD, D_EMB), jnp.float32),
            jax.ShapeDtypeStruct((NPAD, D_EMB), jnp.float32),
        ],
    )(agg1, h0s, dinv_col, W1, b1.reshape(1, D_H))

    agg2a = _agg_kernel(h1s_a, srcp, dstp, cnt, zc)
    agg2b = _agg_kernel(h1s_b, srcp, dstp, cnt, zc)

    wlin_pad = jnp.zeros((D_H, 128), jnp.float32).at[:, :4].set(Wlin)
    blin_pad = jnp.zeros((1, 128), jnp.float32).at[0, :4].set(blin)
    out_pad = pl.pallas_call(
        _mm2_body,
        grid=(NBLK,),
        in_specs=[
            pl.BlockSpec((BM, D_EMB), lambda i: (i, 0)),
            pl.BlockSpec((BM, D_EMB), lambda i: (i, 0)),
            pl.BlockSpec((BM, D_EMB), lambda i: (i, 0)),
            pl.BlockSpec((BM, D_EMB), lambda i: (i, 0)),
            pl.BlockSpec((BM, 1), lambda i: (i, 0)),
            pl.BlockSpec((1, BM), lambda i: (0, i)),
            pl.BlockSpec((D_H, D_H), lambda i: (0, 0)),
            pl.BlockSpec((1, D_H), lambda i: (0, 0)),
            pl.BlockSpec((D_H, 128), lambda i: (0, 0)),
            pl.BlockSpec((1, 128), lambda i: (0, 0)),
        ],
        out_specs=pl.BlockSpec((NG, 128), lambda i: (0, 0)),
        out_shape=jax.ShapeDtypeStruct((NG, 128), jnp.float32),
        scratch_shapes=[
            pltpu.VMEM((NG, D_H), jnp.float32),
            pltpu.VMEM((NG, 1), jnp.float32),
        ],
    )(agg2a, agg2b, h1s_a, h1s_b, dinv_col, batch_row, W2, b2.reshape(1, D_H),
      wlin_pad, blin_pad)

    return out_pad[:, :4]


# speedup vs baseline: 23.5783x; 1.0487x over previous
import functools

import jax
import jax.numpy as jnp
from jax import lax
from jax.experimental import pallas as pl
from jax.experimental.pallas import tpu as pltpu
from jax.experimental.pallas import tpu_sc as plsc

N = 50000
NPAD = 50176
NHALF = NPAD // 2
RPT = NPAD // 32
ZROWS = NPAD - N
E = 800000
EPAD = 819200
EPT = EPAD // 16
W = 128
CH = 2048
WPC = CH // W
D_EMB = 64
D_H = 128
VOCAB = 1024
NG = 256
BM = 512
NBLK = NPAD // BM

_mesh = plsc.VectorSubcoreMesh(core_axis_name="c", subcore_axis_name="s")
_sc_params = pltpu.CompilerParams(needs_layout_passes=False)
_sc_params_untiled = pltpu.CompilerParams(needs_layout_passes=False,
                                          use_tc_tiling_on_sc=False,
                                          internal_scratch_in_bytes=1 << 16)


def _part_body(src_h, dst_h, zrow_h, srcp_h, dstp_h, cnt_h, deg_h,
               srcc, dstc, degidx, degval, cbuf, zbuf, ssta, dsta, acc):
    sc = lax.axis_index("c")
    t = lax.axis_index("s")
    wid = t * 2 + sc
    lo = sc * NHALF
    pltpu.sync_copy(zrow_h, zbuf)
    pltpu.sync_copy(zbuf, acc.at[pl.ds(t * RPT, RPT)])
    plsc.subcore_barrier()
    iota = lax.iota(jnp.int32, 16)

    def chunk(cidx, off):
        cbase = t * EPT + cidx * CH
        pltpu.sync_copy(src_h.at[pl.ds(cbase, CH)], srcc)
        pltpu.sync_copy(dst_h.at[pl.ds(cbase, CH)], dstc)

        def win(w, off):
            def grp(g, off):
                o = w * W + g * 16
                sv = srcc[pl.ds(o, 16)]
                dv = dstc[pl.ds(o, 16)]
                ev = cbase + o + iota
                m = (dv >= lo) & (dv < lo + NHALF) & (ev < E)
                dloc = dv - lo
                plsc.store_compressed(ssta.at[pl.ds(off, 16)], sv, mask=m)
                plsc.store_compressed(dsta.at[pl.ds(off, 16)], dloc, mask=m)
                degidx[pl.ds(g * 16, 16)] = jnp.where(m, dloc, ev & 16383)
                degval[pl.ds(g * 16, 16)] = jnp.where(
                    m, jnp.float32(1.0), jnp.float32(0.0))
                return off + jnp.sum(m.astype(jnp.int32))

            off = lax.fori_loop(0, W // 16, grp, off)
            pltpu.sync_copy(degval, acc.at[degidx], add=True)
            return off

        return lax.fori_loop(0, WPC, win, off)

    off = lax.fori_loop(0, EPT // CH, chunk, jnp.int32(0))

    for k in range(8):
        ssta[pl.ds(off + k * 16, 16)] = N + iota
        dsta[pl.ds(off + k * 16, 16)] = iota
    cnt128 = ((off + 127) // 128) * 128
    cbuf[...] = jnp.full((16,), 0, jnp.int32) + cnt128
    pltpu.sync_copy(cbuf, cnt_h.at[wid])
    obase = sc * EPAD + t * EPT

    def flush(f, _):
        pltpu.sync_copy(ssta.at[pl.ds(f * CH, CH)],
                        srcp_h.at[pl.ds(obase + f * CH, CH)])
        pltpu.sync_copy(dsta.at[pl.ds(f * CH, CH)],
                        dstp_h.at[pl.ds(obase + f * CH, CH)])
        return 0

    lax.fori_loop(0, (cnt128 + CH - 1) // CH, flush, 0)
    plsc.subcore_barrier()
    pltpu.sync_copy(acc.at[pl.ds(t * RPT, RPT)], zbuf)
    pltpu.sync_copy(zbuf, deg_h.at[pl.ds(lo + t * RPT, RPT)])


_part_kernel = functools.partial(
    pl.kernel,
    out_type=[
        jax.ShapeDtypeStruct((2 * EPAD,), jnp.int32),
        jax.ShapeDtypeStruct((2 * EPAD,), jnp.int32),
        jax.ShapeDtypeStruct((32, 16), jnp.int32),
        jax.ShapeDtypeStruct((NPAD,), jnp.float32),
    ],
    mesh=_mesh,
    compiler_params=_sc_params,
    scratch_types=[
        pltpu.VMEM((CH,), jnp.int32),
        pltpu.VMEM((CH,), jnp.int32),
        pltpu.VMEM((W,), jnp.int32),
        pltpu.VMEM((W,), jnp.float32),
        pltpu.VMEM((16,), jnp.int32),
        pltpu.VMEM((RPT,), jnp.float32),
        pltpu.VMEM((EPT + 128,), jnp.int32),
        pltpu.VMEM((EPT + 128,), jnp.int32),
        pltpu.VMEM_SHARED((NHALF,), jnp.float32),
    ],
)(_part_body)


def _h0s_body(x_h, dinv_h, emb_h, out_h, xw, dw, embv, rows):
    sc = lax.axis_index("c")
    t = lax.axis_index("s")
    wid = t * 2 + sc
    g0 = wid * RPT
    pltpu.sync_copy(emb_h, embv)
    pltpu.sync_copy(x_h.at[pl.ds(g0, RPT)], xw)
    pltpu.sync_copy(dinv_h.at[pl.ds(g0, RPT)], dw)
    iota = lax.iota(jnp.int32, 16)
    half_rows = RPT // 2
    half_grps = half_rows // 16

    def half(hf, _):
        def grp(g, _):
            o = hf * half_rows + g * 16
            xv = xw[pl.ds(o, 16)]
            dv = dw[pl.ds(o, 16)]
            xbase = xv * D_EMB
            rbase = (g * 16 + iota) * D_EMB
            for c in range(D_EMB):
                col = plsc.load_gather(embv, [xbase + c])
                plsc.store_scatter(rows, [rbase + c], col * dv)
            return 0

        lax.fori_loop(0, half_grps, grp, 0)
        pltpu.sync_copy(
            rows, out_h.at[pl.ds((g0 + hf * half_rows) * D_EMB,
                                 half_rows * D_EMB)])
        return 0

    lax.fori_loop(0, 2, half, 0)


_h0s_kernel = functools.partial(
    pl.kernel,
    out_type=jax.ShapeDtypeStruct((NPAD * D_EMB,), jnp.float32),
    mesh=_mesh,
    compiler_params=_sc_params,
    scratch_types=[
        pltpu.VMEM((RPT,), jnp.int32),
        pltpu.VMEM((RPT,), jnp.float32),
        pltpu.VMEM((VOCAB * D_EMB,), jnp.float32),
        pltpu.VMEM((RPT // 2 * D_EMB,), jnp.float32),
    ],
)(_h0s_body)


def _agg_body(table_h, srcp_h, dstp_h, cnt_h, zc_h, out_h,
              srcc, dstc, ssel0, dsel0, ssel1, dsel1, rows0, rows1, zbuf,
              cbuf, acc, gsem0, gsem1, ssem0, ssem1):
    sc = lax.axis_index("c")
    t = lax.axis_index("s")
    wid = t * 2 + sc
    lo = sc * NHALF
    pltpu.sync_copy(zc_h, zbuf)
    for q in range(16):
        pltpu.sync_copy(zbuf, acc.at[pl.ds(t * RPT + q * 98, 98)])
    pltpu.sync_copy(cnt_h.at[wid], cbuf)
    nw = cbuf[pl.ds(0, 16)][0] // W
    ebase = sc * EPAD + t * EPT
    plsc.subcore_barrier()

    def sel_compute(j, ssel, dsel):
        koff = (j % WPC) * W

        def grp(g, _):
            ssel[pl.ds(g * 16, 16)] = srcc[pl.ds(koff + g * 16, 16)]
            dsel[pl.ds(g * 16, 16)] = dstc[pl.ds(koff + g * 16, 16)]
            return 0

        lax.fori_loop(0, W // 16, grp, 0)

    def win(j, _):
        @pl.when(j % WPC == 0)
        def _():
            cbase = ebase + (j // WPC) * CH
            pltpu.sync_copy(srcp_h.at[pl.ds(cbase, CH)], srcc)
            pltpu.sync_copy(dstp_h.at[pl.ds(cbase, CH)], dstc)

        def step(ssel_a, dsel_a, rows_a, gsem_a, ssem_a,
                 ssel_b, dsel_b, rows_b, gsem_b, ssem_b):
            @pl.when(j >= 2)
            def _():
                pltpu.make_async_copy(rows_a, acc.at[dsel_a], ssem_a).wait()

            sel_compute(j, ssel_a, dsel_a)
            pltpu.async_copy(table_h.at[ssel_a], rows_a, gsem_a)

            @pl.when(j >= 1)
            def _():
                pltpu.make_async_copy(table_h.at[ssel_b], rows_b, gsem_b).wait()
                pltpu.async_copy(rows_b, acc.at[dsel_b], ssem_b, add=True)

        @pl.when(j % 2 == 0)
        def _():
            step(ssel0, dsel0, rows0, gsem0, ssem0,
                 ssel1, dsel1, rows1, gsem1, ssem1)

        @pl.when(j % 2 == 1)
        def _():
            step(ssel1, dsel1, rows1, gsem1, ssem1,
                 ssel0, dsel0, rows0, gsem0, ssem0)

        return 0

    lax.fori_loop(0, nw, win, 0)
    blast = (nw - 1) % 2

    @pl.when((nw >= 1) & (blast == 0))
    def _():
        pltpu.make_async_copy(table_h.at[ssel0], rows0, gsem0).wait()
        pltpu.async_copy(rows0, acc.at[dsel0], ssem0, add=True)

    @pl.when((nw >= 1) & (blast == 1))
    def _():
        pltpu.make_async_copy(table_h.at[ssel1], rows1, gsem1).wait()
        pltpu.async_copy(rows1, acc.at[dsel1], ssem1, add=True)

    @pl.when((nw >= 2) & (blast == 0))
    def _():
        pltpu.make_async_copy(rows1, acc.at[dsel1], ssem1).wait()

    @pl.when((nw >= 2) & (blast == 1))
    def _():
        pltpu.make_async_copy(rows0, acc.at[dsel0], ssem0).wait()

    @pl.when((nw >= 1) & (blast == 0))
    def _():
        pltpu.make_async_copy(rows0, acc.at[dsel0], ssem0).wait()

    @pl.when((nw >= 1) & (blast == 1))
    def _():
        pltpu.make_async_copy(rows1, acc.at[dsel1], ssem1).wait()

    plsc.subcore_barrier()
    for q in range(16):
        pltpu.sync_copy(acc.at[pl.ds(t * RPT + q * 98, 98)], zbuf)
        pltpu.sync_copy(zbuf, out_h.at[pl.ds(lo + t * RPT + q * 98, 98)])


_agg_kernel = functools.partial(
    pl.kernel,
    out_type=jax.ShapeDtypeStruct((NPAD, D_EMB), jnp.float32),
    mesh=_mesh,
    compiler_params=_sc_params_untiled,
    scratch_types=[
        pltpu.VMEM((CH,), jnp.int32),
        pltpu.VMEM((CH,), jnp.int32),
        pltpu.VMEM((W,), jnp.int32),
        pltpu.VMEM((W,), jnp.int32),
        pltpu.VMEM((W,), jnp.int32),
        pltpu.VMEM((W,), jnp.int32),
        pltpu.VMEM((W, D_EMB), jnp.float32),
        pltpu.VMEM((W, D_EMB), jnp.float32),
        pltpu.VMEM((98, D_EMB), jnp.float32),
        pltpu.VMEM((16,), jnp.int32),
        pltpu.VMEM_SHARED((NHALF, D_EMB), jnp.float32),
        pltpu.SemaphoreType.DMA,
        pltpu.SemaphoreType.DMA,
        pltpu.SemaphoreType.DMA,
        pltpu.SemaphoreType.DMA,
    ],
)(_agg_body)


def _dinv_body(deg_ref, out_ref):
    d = deg_ref[...]
    i0 = lax.broadcasted_iota(jnp.int32, d.shape, 0)
    i1 = lax.broadcasted_iota(jnp.int32, d.shape, 1)
    row = i0 * 128 + i1
    out_ref[...] = jnp.where(row < N, lax.rsqrt(d + 1.0), 0.0)


def _mm1_body(a_ref, hs_ref, dv_ref, w_ref, b_ref, oa_ref, ob_ref):
    a = (a_ref[...] + hs_ref[...]) * dv_ref[...]
    h = jnp.dot(a.astype(jnp.bfloat16), w_ref[...].astype(jnp.bfloat16),
                preferred_element_type=jnp.float32) + b_ref[...]
    h = jnp.maximum(h, 0.0) * dv_ref[...]
    oa_ref[...] = h[:, :D_EMB]
    ob_ref[...] = h[:, D_EMB:]


def _mm2_body(a_ref, b_ref, ha_ref, hb_ref, dv_ref, bt_ref, w2_ref, bb2_ref,
              wl_ref, bl_ref, out_ref, psum, csum):
    i = pl.program_id(0)

    @pl.when(i == 0)
    def _():
        psum[...] = jnp.zeros_like(psum)
        csum[...] = jnp.zeros_like(csum)

    a = (jnp.concatenate([a_ref[...] + ha_ref[...],
                          b_ref[...] + hb_ref[...]], axis=1)) * dv_ref[...]
    h = jnp.dot(a.astype(jnp.bfloat16), w2_ref[...].astype(jnp.bfloat16),
                preferred_element_type=jnp.float32) + bb2_ref[...]
    h = jnp.maximum(h, 0.0)
    bt = bt_ref[...]
    oh = (lax.broadcasted_iota(jnp.int32, (NG, BM), 0) == bt).astype(jnp.bfloat16)
    psum[...] += jnp.dot(oh, h.astype(jnp.bfloat16),
                         preferred_element_type=jnp.float32)
    csum[...] += jnp.sum(oh.astype(jnp.float32), axis=1, keepdims=True)

    @pl.when(i == NBLK - 1)
    def _():
        pooled = psum[...] / jnp.maximum(csum[...], 1.0)
        out_ref[...] = jnp.dot(pooled, wl_ref[...],
                               preferred_element_type=jnp.float32, precision=lax.Precision.HIGHEST) + bl_ref[...]


def kernel(x, edge_index, batch, emb, W1, b1, W2, b2, Wlin, blin):
    x = x.astype(jnp.int32)
    edge_index = edge_index.astype(jnp.int32)
    batch = batch.astype(jnp.int32)

    ei = jnp.full((2, EPAD), NPAD - 1, jnp.int32).at[:, :E].set(edge_index)
    srcf = ei[0]
    dstf = ei[1]
    x_p = jnp.zeros((NPAD,), jnp.int32).at[:N].set(x)
    batch_row = jnp.full((1, NPAD), 1 << 20, jnp.int32).at[0, :N].set(batch)
    zrow = jnp.zeros((RPT,), jnp.float32)
    zc = jnp.zeros((98, D_EMB), jnp.float32)

    srcp, dstp, cnt, deg = _part_kernel(srcf, dstf, zrow)
    dinv2d = pl.pallas_call(
        _dinv_body,
        out_shape=jax.ShapeDtypeStruct((NPAD // 128, 128), jnp.float32),
    )(deg.reshape(NPAD // 128, 128))
    dinv = dinv2d.reshape(NPAD)
    dinv_col = dinv.reshape(NPAD, 1)

    h0s = _h0s_kernel(x_p, dinv, emb.reshape(-1)).reshape(NPAD, D_EMB)

    agg1 = _agg_kernel(h0s, srcp, dstp, cnt, zc)

    h1s_a, h1s_b = pl.pallas_call(
        _mm1_body,
        grid=(NBLK,),
        in_specs=[
            pl.BlockSpec((BM, D_EMB), lambda i: (i, 0)),
            pl.BlockSpec((BM, D_EMB), lambda i: (i, 0)),
            pl.BlockSpec((BM, 1), lambda i: (i, 0)),
            pl.BlockSpec((D_EMB, D_H), lambda i: (0, 0)),
            pl.BlockSpec((1, D_H), lambda i: (0, 0)),
        ],
        out_specs=[
            pl.BlockSpec((BM, D_EMB), lambda i: (i, 0)),
            pl.BlockSpec((BM, D_EMB), lambda i: (i, 0)),
        ],
        out_shape=[
            jax.ShapeDtypeStruct((NPAD, D_EMB), jnp.float32),
            jax.ShapeDtypeStruct((NPAD, D_EMB), jnp.float32),
        ],
    )(agg1, h0s, dinv_col, W1, b1.reshape(1, D_H))

    agg2a = _agg_kernel(h1s_a, srcp, dstp, cnt, zc)
    agg2b = _agg_kernel(h1s_b, srcp, dstp, cnt, zc)

    wlin_pad = jnp.zeros((D_H, 128), jnp.float32).at[:, :4].set(Wlin)
    blin_pad = jnp.zeros((1, 128), jnp.float32).at[0, :4].set(blin)
    out_pad = pl.pallas_call(
        _mm2_body,
        grid=(NBLK,),
        in_specs=[
            pl.BlockSpec((BM, D_EMB), lambda i: (i, 0)),
            pl.BlockSpec((BM, D_EMB), lambda i: (i, 0)),
            pl.BlockSpec((BM, D_EMB), lambda i: (i, 0)),
            pl.BlockSpec((BM, D_EMB), lambda i: (i, 0)),
            pl.BlockSpec((BM, 1), lambda i: (i, 0)),
            pl.BlockSpec((1, BM), lambda i: (0, i)),
            pl.BlockSpec((D_H, D_H), lambda i: (0, 0)),
            pl.BlockSpec((1, D_H), lambda i: (0, 0)),
            pl.BlockSpec((D_H, 128), lambda i: (0, 0)),
            pl.BlockSpec((1, 128), lambda i: (0, 0)),
        ],
        out_specs=pl.BlockSpec((NG, 128), lambda i: (0, 0)),
        out_shape=jax.ShapeDtypeStruct((NG, 128), jnp.float32),
        scratch_shapes=[
            pltpu.VMEM((NG, D_H), jnp.float32),
            pltpu.VMEM((NG, 1), jnp.float32),
        ],
    )(agg2a, agg2b, h1s_a, h1s_b, dinv_col, batch_row, W2, b2.reshape(1, D_H),
      wlin_pad, blin_pad)

    return out_pad[:, :4]
